# Initial kernel scaffold; baseline (speedup 1.0000x reference)
import jax
import jax.numpy as jnp
from jax.experimental import pallas as pl


def kernel(input_ids, token_type_ids, W_word, W_pos, W_type, gamma, beta):
    raise NotImplementedError("write your pallas kernel here")


# speedup vs baseline: 7.2688x; 7.2688x over previous
import functools

import jax
import jax.numpy as jnp
from jax import lax
from jax.experimental import pallas as pl
from jax.experimental.pallas import tpu as pltpu
from jax.experimental.pallas import tpu_sc as plsc

B = 1024
S = 200
HIDDEN = 128
EPS = 1e-12

N_TOK = B * S
NC, NS = 2, 16
NW = NC * NS
IDS_COLS = 128
IDS_ROWS = N_TOK // IDS_COLS
ROWS_PER_W = IDS_ROWS // NW
CHUNK_ROWS = 2
N_CHUNKS = ROWS_PER_W // CHUNK_ROWS
CHUNK_TOK = CHUNK_ROWS * IDS_COLS


def _sc_gather(table, ids2):
    mesh = plsc.VectorSubcoreMesh(core_axis_name="c", subcore_axis_name="s")

    @functools.partial(
        pl.kernel,
        mesh=mesh,
        out_type=jax.ShapeDtypeStruct((N_TOK, HIDDEN), jnp.float32),
        scratch_types=[
            pltpu.VMEM((CHUNK_ROWS, IDS_COLS), jnp.int32),
            pltpu.VMEM((CHUNK_TOK, HIDDEN), jnp.float32),
            pltpu.SemaphoreType.DMA,
        ],
    )
    def gather_kernel(table_hbm, ids_hbm, out_hbm, idx_v, rows_v, sem):
        wid = lax.axis_index("s") * NC + lax.axis_index("c")
        row_base = wid * ROWS_PER_W
        tok_base = row_base * IDS_COLS

        def body(c, carry):
            pltpu.sync_copy(ids_hbm.at[pl.ds(row_base + c * CHUNK_ROWS,
                                             CHUNK_ROWS)], idx_v)
            handles = []
            for j in range(CHUNK_ROWS):
                handles.append(pltpu.async_copy(
                    table_hbm.at[idx_v.at[j]],
                    rows_v.at[pl.ds(j * IDS_COLS, IDS_COLS)],
                    sem))
            for h in handles:
                h.wait()
            pltpu.sync_copy(
                rows_v,
                out_hbm.at[pl.ds(tok_base + c * CHUNK_TOK, CHUNK_TOK)])
            return carry

        lax.fori_loop(0, N_CHUNKS, body, 0)

    return gather_kernel(table, ids2)


TOK_BLK = 1600


def _ln_body(g_ref, tt_ref, pos_ref, type_ref, gam_ref, bet_ref, o_ref):
    x = g_ref[...]
    t0 = type_ref[0:1, :]
    dt = type_ref[1:2, :] - t0
    x = x + pos_ref[...] + t0 + tt_ref[...] * dt
    mean = jnp.mean(x, axis=1, keepdims=True)
    xc = x - mean
    var = jnp.mean(xc * xc, axis=1, keepdims=True)
    o_ref[...] = (xc / jnp.sqrt(var + EPS)) * gam_ref[...] + bet_ref[...]


def _tc_add_ln(gathered, ttf, pos_tile, W_type, gamma, beta):
    n_blk = N_TOK // TOK_BLK
    return pl.pallas_call(
        _ln_body,
        grid=(n_blk,),
        in_specs=[
            pl.BlockSpec((TOK_BLK, HIDDEN), lambda i: (i, 0)),
            pl.BlockSpec((TOK_BLK, 1), lambda i: (i, 0)),
            pl.BlockSpec((TOK_BLK, HIDDEN), lambda i: (0, 0)),
            pl.BlockSpec((2, HIDDEN), lambda i: (0, 0)),
            pl.BlockSpec((1, HIDDEN), lambda i: (0, 0)),
            pl.BlockSpec((1, HIDDEN), lambda i: (0, 0)),
        ],
        out_specs=pl.BlockSpec((TOK_BLK, HIDDEN), lambda i: (i, 0)),
        out_shape=jax.ShapeDtypeStruct((N_TOK, HIDDEN), jnp.float32),
    )(gathered, ttf, pos_tile, W_type, gamma, beta)


def kernel(input_ids, token_type_ids, W_word, W_pos, W_type, gamma, beta):
    ids2 = input_ids.astype(jnp.int32).reshape(IDS_ROWS, IDS_COLS)
    gathered = _sc_gather(W_word, ids2)

    ttf = token_type_ids.astype(jnp.float32).reshape(N_TOK, 1)
    pos_tile = jnp.tile(W_pos[:S], (TOK_BLK // S, 1))
    out = _tc_add_ln(gathered, ttf, pos_tile, W_type,
                     gamma.reshape(1, HIDDEN), beta.reshape(1, HIDDEN))
    return out.reshape(B, S, HIDDEN)


# speedup vs baseline: 9.2171x; 1.2680x over previous
import functools

import jax
import jax.numpy as jnp
from jax import lax
from jax.experimental import pallas as pl
from jax.experimental.pallas import tpu as pltpu
from jax.experimental.pallas import tpu_sc as plsc

B = 1024
S = 200
HIDDEN = 128
EPS = 1e-12

N_TOK = B * S
NC, NS = 2, 16
NW = NC * NS
IDS_COLS = 128
IDS_ROWS = N_TOK // IDS_COLS
ROWS_PER_W = IDS_ROWS // NW
CHUNK_ROWS = 2
N_CHUNKS = ROWS_PER_W // CHUNK_ROWS
CHUNK_TOK = CHUNK_ROWS * IDS_COLS


def _sc_gather(table, ids2):
    mesh = plsc.VectorSubcoreMesh(core_axis_name="c", subcore_axis_name="s")

    @functools.partial(
        pl.kernel,
        mesh=mesh,
        out_type=jax.ShapeDtypeStruct((N_TOK, HIDDEN), jnp.float32),
        scratch_types=[
            pltpu.VMEM((CHUNK_ROWS, IDS_COLS), jnp.int32),
            pltpu.VMEM((CHUNK_TOK, HIDDEN), jnp.float32),
            pltpu.SemaphoreType.DMA,
        ],
    )
    def gather_kernel(table_hbm, ids_hbm, out_hbm, idx_v, rows_v, sem):
        wid = lax.axis_index("s") * NC + lax.axis_index("c")
        row_base = wid * ROWS_PER_W
        tok_base = row_base * IDS_COLS

        def body(c, carry):
            pltpu.sync_copy(ids_hbm.at[pl.ds(row_base + c * CHUNK_ROWS,
                                             CHUNK_ROWS)], idx_v)
            handles = []
            for j in range(CHUNK_ROWS):
                handles.append(pltpu.async_copy(
                    table_hbm.at[idx_v.at[j]],
                    rows_v.at[pl.ds(j * IDS_COLS, IDS_COLS)],
                    sem))
            for h in handles:
                h.wait()
            pltpu.sync_copy(
                rows_v,
                out_hbm.at[pl.ds(tok_base + c * CHUNK_TOK, CHUNK_TOK)])
            return carry

        lax.fori_loop(0, N_CHUNKS, body, 0)

    return gather_kernel(table, ids2)


TOK_BLK = 3200
TT_ROWS = TOK_BLK // 128


def _ln_body(g_ref, tt_ref, pos_ref, type_ref, gam_ref, bet_ref, o_ref):
    x = g_ref[...] + pos_ref[...]
    t0 = type_ref[0:1, :]
    dt = type_ref[1:2, :] - t0
    t2t = jnp.transpose(tt_ref[0])
    term = jnp.concatenate(
        [t2t[:, r:r + 1] * dt for r in range(TT_ROWS)], axis=0)
    x = x + t0 + term
    mean = jnp.mean(x, axis=1, keepdims=True)
    xc = x - mean
    var = jnp.mean(xc * xc, axis=1, keepdims=True)
    o_ref[...] = (xc / jnp.sqrt(var + EPS)) * gam_ref[...] + bet_ref[...]


def _tc_add_ln(gathered, tt_packed, pos_tile, W_type, gamma, beta):
    n_blk = N_TOK // TOK_BLK
    return pl.pallas_call(
        _ln_body,
        grid=(n_blk,),
        in_specs=[
            pl.BlockSpec((TOK_BLK, HIDDEN), lambda i: (i, 0)),
            pl.BlockSpec((1, TT_ROWS, 128), lambda i: (i, 0, 0)),
            pl.BlockSpec((TOK_BLK, HIDDEN), lambda i: (0, 0)),
            pl.BlockSpec((2, HIDDEN), lambda i: (0, 0)),
            pl.BlockSpec((1, HIDDEN), lambda i: (0, 0)),
            pl.BlockSpec((1, HIDDEN), lambda i: (0, 0)),
        ],
        out_specs=pl.BlockSpec((TOK_BLK, HIDDEN), lambda i: (i, 0)),
        out_shape=jax.ShapeDtypeStruct((N_TOK, HIDDEN), jnp.float32),
    )(gathered, tt_packed, pos_tile, W_type, gamma, beta)


def kernel(input_ids, token_type_ids, W_word, W_pos, W_type, gamma, beta):
    ids2 = input_ids.astype(jnp.int32).reshape(IDS_ROWS, IDS_COLS)
    gathered = _sc_gather(W_word, ids2)

    tt_packed = token_type_ids.astype(jnp.float32).reshape(
        N_TOK // TOK_BLK, TT_ROWS, 128)
    pos_tile = jnp.tile(W_pos[:S], (TOK_BLK // S, 1))
    out = _tc_add_ln(gathered, tt_packed, pos_tile, W_type,
                     gamma.reshape(1, HIDDEN), beta.reshape(1, HIDDEN))
    return out.reshape(B, S, HIDDEN)


# speedup vs baseline: 11.0776x; 1.2019x over previous
import functools

import jax
import jax.numpy as jnp
from jax import lax
from jax.experimental import pallas as pl
from jax.experimental.pallas import tpu as pltpu
from jax.experimental.pallas import tpu_sc as plsc

B = 1024
S = 200
HIDDEN = 128
EPS = 1e-12

N_TOK = B * S
NC, NS = 2, 16
NW = NC * NS
IDS_COLS = 128
IDS_ROWS = N_TOK // IDS_COLS
N_SLAB = 2
SLAB_ROWS = IDS_ROWS // N_SLAB
SLAB_TOK = SLAB_ROWS * IDS_COLS
ROWS_PER_W = SLAB_ROWS // NW
CHUNK_ROWS = 5
N_CHUNKS = ROWS_PER_W // CHUNK_ROWS
CHUNK_TOK = CHUNK_ROWS * IDS_COLS


def _sc_gather(table, ids3):
    mesh = plsc.VectorSubcoreMesh(core_axis_name="c", subcore_axis_name="s")

    @functools.partial(
        pl.kernel,
        mesh=mesh,
        out_type=jax.ShapeDtypeStruct((SLAB_TOK, HIDDEN), jnp.float32),
        scratch_types=[
            pltpu.VMEM((ROWS_PER_W, IDS_COLS), jnp.int32),
            pltpu.VMEM((CHUNK_TOK, HIDDEN), jnp.float32),
            pltpu.SemaphoreType.DMA,
        ],
    )
    def gather_kernel(table_hbm, ids_hbm, out_hbm, idx_v, rows_v, sem):
        wid = lax.axis_index("s") * NC + lax.axis_index("c")
        tok_base = wid * ROWS_PER_W * IDS_COLS
        pltpu.sync_copy(ids_hbm.at[wid], idx_v)

        def body(c, carry):
            handles = []
            for j in range(CHUNK_ROWS):
                handles.append(pltpu.async_copy(
                    table_hbm.at[idx_v.at[c * CHUNK_ROWS + j]],
                    rows_v.at[pl.ds(j * IDS_COLS, IDS_COLS)],
                    sem))
            for h in handles:
                h.wait()
            pltpu.sync_copy(
                rows_v,
                out_hbm.at[pl.ds(tok_base + c * CHUNK_TOK, CHUNK_TOK)])
            return carry

        lax.fori_loop(0, N_CHUNKS, body, 0)

    return gather_kernel(table, ids3)


TOK_BLK = 3200
TT_ROWS = TOK_BLK // 128


def _ln_body(g_ref, tt_ref, pos_ref, type_ref, gam_ref, bet_ref, o_ref):
    x = g_ref[...] + pos_ref[...]
    t0 = type_ref[0:1, :]
    dt = type_ref[1:2, :] - t0
    t2t = jnp.transpose(tt_ref[0])
    term = jnp.concatenate(
        [t2t[:, r:r + 1] * dt for r in range(TT_ROWS)], axis=0)
    x = x + t0 + term
    mean = jnp.mean(x, axis=1, keepdims=True)
    xc = x - mean
    var = jnp.mean(xc * xc, axis=1, keepdims=True)
    o_ref[...] = (xc / jnp.sqrt(var + EPS)) * gam_ref[...] + bet_ref[...]


def _ln_body_alias(g_ref, tt_ref, pos_ref, type_ref, gam_ref, bet_ref,
                   prev_ref, o_ref):
    _ln_body(g_ref, tt_ref, pos_ref, type_ref, gam_ref, bet_ref, o_ref)


SLAB_BLKS = SLAB_TOK // TOK_BLK


def _tc_add_ln(slab, gathered, tt_packed, pos_tile, W_type, gamma, beta,
               prev_out):
    base = slab * SLAB_BLKS
    in_specs = [
        pl.BlockSpec((TOK_BLK, HIDDEN), lambda i: (i, 0)),
        pl.BlockSpec((1, TT_ROWS, 128), lambda i, b=base: (b + i, 0, 0)),
        pl.BlockSpec((TOK_BLK, HIDDEN), lambda i: (0, 0)),
        pl.BlockSpec((2, HIDDEN), lambda i: (0, 0)),
        pl.BlockSpec((1, HIDDEN), lambda i: (0, 0)),
        pl.BlockSpec((1, HIDDEN), lambda i: (0, 0)),
    ]
    args = [gathered, tt_packed, pos_tile, W_type, gamma, beta]
    kwargs = {}
    body = _ln_body
    if prev_out is not None:
        in_specs.append(pl.BlockSpec(memory_space=pl.ANY))
        args.append(prev_out)
        kwargs["input_output_aliases"] = {6: 0}
        body = _ln_body_alias
    return pl.pallas_call(
        body,
        grid=(SLAB_BLKS,),
        in_specs=in_specs,
        out_specs=pl.BlockSpec((TOK_BLK, HIDDEN),
                               lambda i, b=base: (b + i, 0)),
        out_shape=jax.ShapeDtypeStruct((N_TOK, HIDDEN), jnp.float32),
        **kwargs,
    )(*args)


def kernel(input_ids, token_type_ids, W_word, W_pos, W_type, gamma, beta):
    ids2 = input_ids.astype(jnp.int32).reshape(IDS_ROWS, IDS_COLS)
    tt_packed = token_type_ids.astype(jnp.float32).reshape(
        N_TOK // TOK_BLK, TT_ROWS, 128)
    pos_tile = jnp.tile(W_pos[:S], (TOK_BLK // S, 1))
    gam = gamma.reshape(1, HIDDEN)
    bet = beta.reshape(1, HIDDEN)

    ids4 = ids2.reshape(N_SLAB, NW, ROWS_PER_W, IDS_COLS)
    slabs = [_sc_gather(W_word, ids4[s]) for s in range(N_SLAB)]
    out = None
    for s in range(N_SLAB):
        out = _tc_add_ln(s, slabs[s], tt_packed, pos_tile, W_type,
                         gam, bet, out)
    return out.reshape(B, S, HIDDEN)


# speedup vs baseline: 11.6637x; 1.0529x over previous
import functools

import jax
import jax.numpy as jnp
from jax import lax
from jax.experimental import pallas as pl
from jax.experimental.pallas import tpu as pltpu
from jax.experimental.pallas import tpu_sc as plsc

B = 1024
S = 200
HIDDEN = 128
EPS = 1e-12

N_TOK = B * S
NC, NS = 2, 16
NW = NC * NS
IDS_COLS = 128
IDS_ROWS = N_TOK // IDS_COLS
N_SLAB = 2
SLAB_ROWS = IDS_ROWS // N_SLAB
SLAB_TOK = SLAB_ROWS * IDS_COLS
ROWS_PER_W = SLAB_ROWS // NW
CHUNK_ROWS = 5
N_CHUNKS = ROWS_PER_W // CHUNK_ROWS
CHUNK_TOK = CHUNK_ROWS * IDS_COLS


def _sc_gather(table, ids3):
    mesh = plsc.VectorSubcoreMesh(core_axis_name="c", subcore_axis_name="s")

    @functools.partial(
        pl.kernel,
        mesh=mesh,
        out_type=jax.ShapeDtypeStruct((SLAB_TOK, HIDDEN), jnp.float32),
        scratch_types=[
            pltpu.VMEM((ROWS_PER_W, IDS_COLS), jnp.int32),
            pltpu.VMEM((CHUNK_TOK, HIDDEN), jnp.float32),
            pltpu.SemaphoreType.DMA,
        ],
    )
    def gather_kernel(table_hbm, ids_hbm, out_hbm, idx_v, rows_v, sem):
        wid = lax.axis_index("s") * NC + lax.axis_index("c")
        tok_base = wid * ROWS_PER_W * IDS_COLS
        pltpu.sync_copy(ids_hbm.at[wid], idx_v)

        def body(c, carry):
            handles = []
            for j in range(CHUNK_ROWS):
                handles.append(pltpu.async_copy(
                    table_hbm.at[idx_v.at[c * CHUNK_ROWS + j]],
                    rows_v.at[pl.ds(j * IDS_COLS, IDS_COLS)],
                    sem))
            for h in handles:
                h.wait()
            pltpu.sync_copy(
                rows_v,
                out_hbm.at[pl.ds(tok_base + c * CHUNK_TOK, CHUNK_TOK)])
            return carry

        lax.fori_loop(0, N_CHUNKS, body, 0)

    return gather_kernel(table, ids3)


TOK_BLK = 6400
TT_ROWS = TOK_BLK // 128


def _ln_body(g_ref, tt_ref, pos_ref, type_ref, gam_ref, bet_ref, o_ref):
    x = g_ref[...] + pos_ref[...]
    t0 = type_ref[0:1, :]
    dt = type_ref[1:2, :] - t0
    t2t = jnp.transpose(tt_ref[0])
    term = jnp.concatenate(
        [t2t[:, r:r + 1] * dt for r in range(TT_ROWS)], axis=0)
    x = x + t0 + term
    mean = jnp.mean(x, axis=1, keepdims=True)
    xc = x - mean
    var = jnp.mean(xc * xc, axis=1, keepdims=True)
    o_ref[...] = (xc / jnp.sqrt(var + EPS)) * gam_ref[...] + bet_ref[...]


def _ln_body_alias(g_ref, tt_ref, pos_ref, type_ref, gam_ref, bet_ref,
                   prev_ref, o_ref):
    _ln_body(g_ref, tt_ref, pos_ref, type_ref, gam_ref, bet_ref, o_ref)


SLAB_BLKS = SLAB_TOK // TOK_BLK


def _tc_add_ln(slab, gathered, tt_packed, pos_tile, W_type, gamma, beta,
               prev_out):
    base = slab * SLAB_BLKS
    in_specs = [
        pl.BlockSpec((TOK_BLK, HIDDEN), lambda i: (i, 0)),
        pl.BlockSpec((1, TT_ROWS, 128), lambda i, b=base: (b + i, 0, 0)),
        pl.BlockSpec((TOK_BLK, HIDDEN), lambda i: (0, 0)),
        pl.BlockSpec((2, HIDDEN), lambda i: (0, 0)),
        pl.BlockSpec((1, HIDDEN), lambda i: (0, 0)),
        pl.BlockSpec((1, HIDDEN), lambda i: (0, 0)),
    ]
    args = [gathered, tt_packed, pos_tile, W_type, gamma, beta]
    kwargs = {}
    body = _ln_body
    if prev_out is not None:
        in_specs.append(pl.BlockSpec(memory_space=pl.ANY))
        args.append(prev_out)
        kwargs["input_output_aliases"] = {6: 0}
        body = _ln_body_alias
    return pl.pallas_call(
        body,
        grid=(SLAB_BLKS,),
        in_specs=in_specs,
        out_specs=pl.BlockSpec((TOK_BLK, HIDDEN),
                               lambda i, b=base: (b + i, 0)),
        out_shape=jax.ShapeDtypeStruct((N_TOK, HIDDEN), jnp.float32),
        **kwargs,
    )(*args)


def kernel(input_ids, token_type_ids, W_word, W_pos, W_type, gamma, beta):
    ids2 = input_ids.astype(jnp.int32).reshape(IDS_ROWS, IDS_COLS)
    tt_packed = token_type_ids.astype(jnp.float32).reshape(
        N_TOK // TOK_BLK, TT_ROWS, 128)
    pos_tile = jnp.tile(W_pos[:S], (TOK_BLK // S, 1))
    gam = gamma.reshape(1, HIDDEN)
    bet = beta.reshape(1, HIDDEN)

    ids4 = ids2.reshape(N_SLAB, NW, ROWS_PER_W, IDS_COLS)
    slabs = [_sc_gather(W_word, ids4[s]) for s in range(N_SLAB)]
    out = None
    for s in range(N_SLAB):
        out = _tc_add_ln(s, slabs[s], tt_packed, pos_tile, W_type,
                         gam, bet, out)
    return out.reshape(B, S, HIDDEN)


# speedup vs baseline: 12.0746x; 1.0352x over previous
import functools

import jax
import jax.numpy as jnp
from jax import lax
from jax.experimental import pallas as pl
from jax.experimental.pallas import tpu as pltpu
from jax.experimental.pallas import tpu_sc as plsc

B = 1024
S = 200
HIDDEN = 128
EPS = 1e-12

N_TOK = B * S
NC, NS = 2, 16
NW = NC * NS
IDS_COLS = 128
IDS_ROWS = N_TOK // IDS_COLS
N_SLAB = 2
SLAB_ROWS = IDS_ROWS // N_SLAB
SLAB_TOK = SLAB_ROWS * IDS_COLS
ROWS_PER_W = SLAB_ROWS // NW
CHUNK_ROWS = 5
N_CHUNKS = ROWS_PER_W // CHUNK_ROWS
CHUNK_TOK = CHUNK_ROWS * IDS_COLS


def _sc_gather(table, ids3):
    mesh = plsc.VectorSubcoreMesh(core_axis_name="c", subcore_axis_name="s")

    @functools.partial(
        pl.kernel,
        mesh=mesh,
        out_type=jax.ShapeDtypeStruct((SLAB_TOK, HIDDEN), jnp.float32),
        scratch_types=[
            pltpu.VMEM((ROWS_PER_W, IDS_COLS), jnp.int32),
            pltpu.VMEM((CHUNK_TOK, HIDDEN), jnp.float32),
            pltpu.SemaphoreType.DMA,
        ],
    )
    def gather_kernel(table_hbm, ids_hbm, out_hbm, idx_v, rows_v, sem):
        wid = lax.axis_index("s") * NC + lax.axis_index("c")
        tok_base = wid * ROWS_PER_W * IDS_COLS
        pltpu.sync_copy(ids_hbm.at[wid], idx_v)

        def body(c, carry):
            handles = []
            for j in range(CHUNK_ROWS):
                handles.append(pltpu.async_copy(
                    table_hbm.at[idx_v.at[c * CHUNK_ROWS + j]],
                    rows_v.at[pl.ds(j * IDS_COLS, IDS_COLS)],
                    sem))
            for h in handles:
                h.wait()
            pltpu.sync_copy(
                rows_v,
                out_hbm.at[pl.ds(tok_base + c * CHUNK_TOK, CHUNK_TOK)])
            return carry

        lax.fori_loop(0, N_CHUNKS, body, 0)

    return gather_kernel(table, ids3)


TOK_BLK = 6400
TT_ROWS = TOK_BLK // 128


def _ln_body(g_ref, tt_ref, pos_ref, type_ref, gam_ref, bet_ref, o_ref):
    x = g_ref[...] + pos_ref[...]
    t0 = type_ref[0:1, :]
    dt = type_ref[1:2, :] - t0
    t2t = jnp.transpose(tt_ref[0])
    term = jnp.concatenate(
        [t2t[:, r:r + 1] * dt for r in range(TT_ROWS)], axis=0)
    x = x + t0 + term
    inv_h = jnp.float32(1.0 / HIDDEN)
    mean = jnp.sum(x, axis=1, keepdims=True) * inv_h
    ex2 = jnp.sum(x * x, axis=1, keepdims=True) * inv_h
    var = ex2 - mean * mean
    inv_std = lax.rsqrt(var + EPS)
    a = inv_std * gam_ref[...]
    o_ref[...] = (x - mean) * a + bet_ref[...]


def _ln_body_alias(g_ref, tt_ref, pos_ref, type_ref, gam_ref, bet_ref,
                   prev_ref, o_ref):
    _ln_body(g_ref, tt_ref, pos_ref, type_ref, gam_ref, bet_ref, o_ref)


SLAB_BLKS = SLAB_TOK // TOK_BLK


def _tc_add_ln(slab, gathered, tt_packed, pos_tile, W_type, gamma, beta,
               prev_out):
    base = slab * SLAB_BLKS
    in_specs = [
        pl.BlockSpec((TOK_BLK, HIDDEN), lambda i: (i, 0)),
        pl.BlockSpec((1, TT_ROWS, 128), lambda i, b=base: (b + i, 0, 0)),
        pl.BlockSpec((TOK_BLK, HIDDEN), lambda i: (0, 0)),
        pl.BlockSpec((2, HIDDEN), lambda i: (0, 0)),
        pl.BlockSpec((1, HIDDEN), lambda i: (0, 0)),
        pl.BlockSpec((1, HIDDEN), lambda i: (0, 0)),
    ]
    args = [gathered, tt_packed, pos_tile, W_type, gamma, beta]
    kwargs = {}
    body = _ln_body
    if prev_out is not None:
        in_specs.append(pl.BlockSpec(memory_space=pl.ANY))
        args.append(prev_out)
        kwargs["input_output_aliases"] = {6: 0}
        body = _ln_body_alias
    return pl.pallas_call(
        body,
        grid=(SLAB_BLKS,),
        in_specs=in_specs,
        out_specs=pl.BlockSpec((TOK_BLK, HIDDEN),
                               lambda i, b=base: (b + i, 0)),
        out_shape=jax.ShapeDtypeStruct((N_TOK, HIDDEN), jnp.float32),
        **kwargs,
    )(*args)


def kernel(input_ids, token_type_ids, W_word, W_pos, W_type, gamma, beta):
    ids2 = input_ids.astype(jnp.int32).reshape(IDS_ROWS, IDS_COLS)
    tt_packed = token_type_ids.astype(jnp.float32).reshape(
        N_TOK // TOK_BLK, TT_ROWS, 128)
    pos_tile = jnp.tile(W_pos[:S], (TOK_BLK // S, 1))
    gam = gamma.reshape(1, HIDDEN)
    bet = beta.reshape(1, HIDDEN)

    ids4 = ids2.reshape(N_SLAB, NW, ROWS_PER_W, IDS_COLS)
    slabs = [_sc_gather(W_word, ids4[s]) for s in range(N_SLAB)]
    out = None
    for s in range(N_SLAB):
        out = _tc_add_ln(s, slabs[s], tt_packed, pos_tile, W_type,
                         gam, bet, out)
    return out.reshape(B, S, HIDDEN)
